# baseline (speedup 1.0000x reference)
import functools

import jax
import jax.numpy as jnp
from jax import lax
from jax.experimental import pallas as pl
from jax.experimental.pallas import tpu as pltpu
from jax.experimental.pallas import tpu_sc as plsc

N = 10000
E = 160000
E_LG = 640000
F = 128

_NC = 2
_NT = 16
_B = 80


@functools.lru_cache(maxsize=None)
def _make_segsum(n_edges, n_out, w, n_slices, src_mul, sl_mul, linear_src):
    ept = n_edges // _NT
    assert n_edges % (_NT * _B) == 0
    nb = ept // _B
    zr = {64: 80, 8: 1000}[w]
    chunk = _NT * zr
    acc_rows = ((n_out + chunk - 1) // chunk) * chunk
    rpt = acc_rows // _NT
    assert rpt % zr == 0
    spc = n_slices // _NC

    mesh = plsc.VectorSubcoreMesh(core_axis_name="c", subcore_axis_name="s")

    scratch = [
        pltpu.VMEM((_B,), jnp.int32),
        pltpu.VMEM((_B,), jnp.int32),
        pltpu.VMEM((_B,), jnp.int32),
        pltpu.VMEM((_B, w), jnp.float32),
        pltpu.VMEM((zr, w), jnp.float32),
        pltpu.VMEM_SHARED((acc_rows, w), jnp.float32),
        pltpu.SemaphoreType.DMA,
    ]

    @functools.partial(
        pl.kernel,
        out_type=jax.ShapeDtypeStruct((n_slices, n_out, w), jnp.float32),
        mesh=mesh,
        scratch_types=scratch,
        compiler_params=pltpu.CompilerParams(use_tc_tiling_on_sc=False),
    )
    def seg(table, src, dst, zeros, out, src_v, dst_v, gi_v, rows_v, z_v,
            acc, sem):
        c = lax.axis_index("c")
        t = lax.axis_index("s")
        t_lo = t * ept
        row_lo = t * rpt
        n_valid = jnp.minimum(jnp.maximum(n_out - row_lo, 0), rpt)
        pltpu.sync_copy(zeros, z_v)

        for j in range(spc):
            sl = c * spc + j

            for q in range(rpt // zr):
                pltpu.sync_copy(z_v, acc.at[pl.ds(row_lo + q * zr, zr), :])
            plsc.subcore_barrier()

            def body(k, carry):
                base = t_lo + k * _B
                if not linear_src:
                    pltpu.sync_copy(src.at[pl.ds(base, _B)], src_v)
                pltpu.sync_copy(dst.at[pl.ds(base, _B)], dst_v)
                for u in range(_B // 16):
                    if linear_src:
                        sv = base + u * 16 + lax.iota(jnp.int32, 16)
                    else:
                        sv = src_v[pl.ds(u * 16, 16)]
                    gi_v[pl.ds(u * 16, 16)] = sv * src_mul + sl * sl_mul
                pltpu.async_copy(table.at[gi_v], rows_v, sem).wait()
                pltpu.sync_copy(rows_v, acc.at[dst_v], add=True)
                return carry

            lax.fori_loop(0, nb, body, 0)
            plsc.subcore_barrier()

            if acc_rows == n_out:
                pltpu.sync_copy(
                    acc.at[pl.ds(row_lo, rpt), :],
                    out.at[sl, pl.ds(row_lo, rpt), :])
            else:
                vrows = n_out - (_NT - 1) * rpt

                @pl.when(n_valid == rpt)
                def _():
                    pltpu.sync_copy(
                        acc.at[pl.ds(row_lo, rpt), :],
                        out.at[sl, pl.ds(row_lo, rpt), :])

                @pl.when(n_valid < rpt)
                def _():
                    pltpu.sync_copy(
                        acc.at[pl.ds(row_lo, vrows), :],
                        out.at[sl, pl.ds(row_lo, vrows), :])

            plsc.subcore_barrier()

    return seg


@functools.lru_cache(maxsize=None)
def _make_gather(n_idx):
    nw = _NC * _NT
    ipw = n_idx // nw
    gb = 40
    assert ipw % gb == 0

    mesh = plsc.VectorSubcoreMesh(core_axis_name="c", subcore_axis_name="s")

    @functools.partial(
        pl.kernel,
        out_type=jax.ShapeDtypeStruct((n_idx, F), jnp.float32),
        mesh=mesh,
        scratch_types=[
            pltpu.VMEM((ipw,), jnp.int32),
            pltpu.VMEM((gb, F), jnp.float32),
            pltpu.SemaphoreType.DMA,
        ],
    )
    def gat(table, idx, out, idx_v, rows_v, sem):
        c = lax.axis_index("c")
        t = lax.axis_index("s")
        wid = t * _NC + c
        base = wid * ipw
        pltpu.sync_copy(idx.at[pl.ds(base, ipw)], idx_v)

        def body(k, carry):
            pltpu.async_copy(table.at[idx_v.at[pl.ds(k * gb, gb)]],
                             rows_v, sem).wait()
            pltpu.sync_copy(rows_v, out.at[pl.ds(base + k * gb, gb)])
            return carry

        lax.fori_loop(0, ipw // gb, body, 0)

    return gat


def _unslice(raw, n_out, w):
    return raw.transpose(1, 0, 2).reshape(n_out, F)


def _d1_body(x_ref, deg_ref, z1_ref, z2_ref, pm_ref, wx, wd, w0, w1, wp,
             bias, pre_ref, s1_ref, s2_ref):
    i = pl.program_id(0)
    xb = x_ref[...]
    a = jnp.dot(xb, wx[...], preferred_element_type=jnp.float32)
    a += jnp.dot(xb * deg_ref[...], wd[...], preferred_element_type=jnp.float32)
    a += jnp.dot(z1_ref[...], w0[...], preferred_element_type=jnp.float32)
    a += jnp.dot(z2_ref[...], w1[...], preferred_element_type=jnp.float32)
    a += jnp.dot(pm_ref[...], wp[...], preferred_element_type=jnp.float32)
    a += bias[...]
    col = lax.broadcasted_iota(jnp.int32, a.shape, 1)
    a = jnp.where(col >= F // 2, jnp.maximum(a, 0.0), a)
    pre_ref[...] = a

    @pl.when(i == 0)
    def _init():
        s1_ref[...] = jnp.zeros_like(s1_ref)
        s2_ref[...] = jnp.zeros_like(s2_ref)

    s1_ref[...] += jnp.sum(a, axis=0, keepdims=True)
    s2_ref[...] += jnp.sum(a * a, axis=0, keepdims=True)


def _d2_body(pre_ref, s1_ref, s2_ref, gw, gb_, out_ref, *, n_rows):
    mean = s1_ref[...] / n_rows
    var = s2_ref[...] / n_rows - mean * mean
    inv = lax.rsqrt(var + 1e-5)
    out_ref[...] = (pre_ref[...] - mean) * inv * gw[...] + gb_[...]


def _dense_side(xx, deg, z1, z2, pm, wx, wd, w0, w1, wp, bias, gw, gb_, bs):
    n_rows = xx.shape[0]
    grid = n_rows // bs
    row = lambda i: (i, 0)
    const = lambda i: (0, 0)
    bspec = pl.BlockSpec((bs, F), row)
    wspec = pl.BlockSpec((F, F), const)
    sspec = pl.BlockSpec((1, F), const)

    pre, s1, s2 = pl.pallas_call(
        _d1_body,
        grid=(grid,),
        in_specs=[bspec, pl.BlockSpec((bs, 1), row), bspec, bspec, bspec,
                  wspec, wspec, wspec, wspec, wspec, sspec],
        out_specs=[bspec, sspec, sspec],
        out_shape=[jax.ShapeDtypeStruct((n_rows, F), jnp.float32),
                   jax.ShapeDtypeStruct((1, F), jnp.float32),
                   jax.ShapeDtypeStruct((1, F), jnp.float32)],
    )(xx, deg, z1, z2, pm, wx, wd, w0, w1, wp, bias)

    out = pl.pallas_call(
        functools.partial(_d2_body, n_rows=float(n_rows)),
        grid=(grid,),
        in_specs=[bspec, sspec, sspec, sspec, sspec],
        out_specs=bspec,
        out_shape=jax.ShapeDtypeStruct((n_rows, F), jnp.float32),
    )(pre, s1, s2, gw, gb_)
    return out


def kernel(x, y, deg_g, deg_lg, pm_pd, edge_index_g, edge_index_lg, params):
    p = params
    src_g, dst_g = edge_index_g[0], edge_index_g[1]
    src_lg, dst_lg = edge_index_lg[0], edge_index_lg[1]
    zeros_g = jnp.zeros((80, 64), jnp.float32)
    zeros_lg = jnp.zeros((1000, 8), jnp.float32)

    seg_g1 = _make_segsum(E, N, 64, 2, 2, 1, False)
    seg_g2 = _make_segsum(E, N, 64, 2, 1, N, False)
    seg_gy = _make_segsum(E, N, 64, 2, 2, 1, True)
    seg_lg1 = _make_segsum(E_LG, E, 8, 16, 16, 1, False)
    seg_lg2 = _make_segsum(E_LG, E, 8, 16, 1, E, False)
    gather = _make_gather(E)

    z1g_r = seg_g1(x.reshape(N * 2, 64), src_g, dst_g, zeros_g)
    z2g_r = seg_g2(z1g_r.reshape(N * 2, 64), src_g, dst_g, zeros_g)
    pmy_r = seg_gy(y.reshape(E * 2, 64), src_g, dst_g, zeros_g)
    z1lg_r = seg_lg1(y.reshape(E * 16, 8), src_lg, dst_lg, zeros_lg)
    z2lg_r = seg_lg2(z1lg_r.reshape(E * 16, 8), src_lg, dst_lg, zeros_lg)
    pmx = gather(x, pm_pd)

    z1g = _unslice(z1g_r, N, 64)
    z2g = _unslice(z2g_r, N, 64)
    pmy = _unslice(pmy_r, N, 64)
    z1lg = _unslice(z1lg_r, E, 8)
    z2lg = _unslice(z2lg_r, E, 8)

    def wT(nm):
        return p[nm + '_w'].T

    bias_x = (p['theta_x_b'] + p['theta_deg_b'] + p['theta_0_b'] +
              p['theta_1_b'] + p['theta_y_b']).reshape(1, F)
    bias_y = (p['gamma_y_b'] + p['gamma_deg_b'] + p['gamma_0_b'] +
              p['gamma_1_b'] + p['gamma_x_b']).reshape(1, F)

    x_out = _dense_side(
        x, deg_g, z1g, z2g, pmy,
        wT('theta_x'), wT('theta_deg'), wT('theta_0'), wT('theta_1'),
        wT('theta_y'), bias_x,
        p['bn_x_w'].reshape(1, F), p['bn_x_b'].reshape(1, F), 1000)
    y_out = _dense_side(
        y, deg_lg, z1lg, z2lg, pmx,
        wT('gamma_y'), wT('gamma_deg'), wT('gamma_0'), wT('gamma_1'),
        wT('gamma_x'), bias_y,
        p['bn_y_w'].reshape(1, F), p['bn_y_b'].reshape(1, F), 2000)
    return (x_out, y_out)


# speedup vs baseline: 3.6461x; 3.6461x over previous
import functools

import jax
import jax.numpy as jnp
from jax import lax
from jax.experimental import pallas as pl
from jax.experimental.pallas import tpu as pltpu
from jax.experimental.pallas import tpu_sc as plsc

N = 10000
E = 160000
E_LG = 640000
F = 128

_NC = 2
_NT = 16
_MB = 128
_NMB = 8
_SB = _MB * _NMB


@functools.lru_cache(maxsize=None)
def _make_segsum(n_edges, n_out, w, n_slices, src_mul, sl_mul, linear_src,
                 src_max):
    ept = n_edges // _NT
    assert n_edges % (_NT * _SB) == 0, n_edges
    nsb = ept // _SB
    zr = {64: 80, 8: 1000}[w]
    chunk = _NT * zr
    acc_rows = ((n_out + chunk - 1) // chunk) * chunk
    rpt = acc_rows // _NT
    assert rpt % zr == 0
    spc = n_slices // _NC

    mesh = plsc.VectorSubcoreMesh(core_axis_name="c", subcore_axis_name="s")

    scratch = [
        pltpu.VMEM((_NMB, _MB), jnp.int32),
        pltpu.VMEM((_NMB, _MB), jnp.int32),
        [pltpu.VMEM((_MB,), jnp.int32) for _ in range(_NMB)],
        [pltpu.VMEM((_MB, w), jnp.float32) for _ in range(_NMB)],
        pltpu.VMEM((zr, w), jnp.float32),
        pltpu.VMEM_SHARED((acc_rows + 8, w), jnp.float32),
        pltpu.SemaphoreType.DMA,
        pltpu.SemaphoreType.DMA,
    ]

    @functools.partial(
        pl.kernel,
        out_type=jax.ShapeDtypeStruct((n_slices, n_out, w), jnp.float32),
        mesh=mesh,
        scratch_types=scratch,
        compiler_params=pltpu.CompilerParams(use_tc_tiling_on_sc=False),
    )
    def seg(table, src2, dst2, zeros, out, src_v, dst_v, gi, rows, z_v,
            acc, gsem, ssem):
        c = lax.axis_index("c")
        t = lax.axis_index("s")
        t_lo = t * (ept // _MB)
        row_lo = t * rpt
        n_valid = jnp.minimum(jnp.maximum(n_out - row_lo, 0), rpt)
        pltpu.sync_copy(zeros, z_v)

        for j in range(spc):
            sl = c * spc + j

            for q in range(rpt // zr):
                pltpu.sync_copy(z_v, acc.at[pl.ds(row_lo + q * zr, zr), :])
            plsc.subcore_barrier()

            def body(k, carry):
                rbase = t_lo + k * _NMB
                if not linear_src:
                    pltpu.sync_copy(src2.at[pl.ds(rbase, _NMB), :], src_v)
                pltpu.sync_copy(dst2.at[pl.ds(rbase, _NMB), :], dst_v)
                for m in range(_NMB):
                    for u in range(_MB // 16):
                        if linear_src:
                            sv = ((rbase + m) * _MB + u * 16 +
                                  lax.iota(jnp.int32, 16))
                            sv = jnp.minimum(sv, src_max)
                        else:
                            sv = src_v[m, pl.ds(u * 16, 16)]
                        gi[m][pl.ds(u * 16, 16)] = sv * src_mul + sl * sl_mul
                gds = [pltpu.async_copy(table.at[gi[m]], rows[m], gsem)
                       for m in range(_NMB)]
                sds = []
                for m in range(_NMB):
                    gds[m].wait()
                    sds.append(pltpu.async_copy(
                        rows[m], acc.at[dst_v.at[m]], ssem, add=True))
                for d in sds:
                    d.wait()
                return carry

            lax.fori_loop(0, nsb, body, 0)
            plsc.subcore_barrier()

            if acc_rows == n_out:
                pltpu.sync_copy(
                    acc.at[pl.ds(row_lo, rpt), :],
                    out.at[sl, pl.ds(row_lo, rpt), :])
            else:
                vrows = n_out - (_NT - 1) * rpt

                @pl.when(n_valid == rpt)
                def _():
                    pltpu.sync_copy(
                        acc.at[pl.ds(row_lo, rpt), :],
                        out.at[sl, pl.ds(row_lo, rpt), :])

                @pl.when(n_valid < rpt)
                def _():
                    pltpu.sync_copy(
                        acc.at[pl.ds(row_lo, vrows), :],
                        out.at[sl, pl.ds(row_lo, vrows), :])

            plsc.subcore_barrier()

    return seg


@functools.lru_cache(maxsize=None)
def _make_gather(n_idx):
    nw = _NC * _NT
    ipw = n_idx // nw
    gb = 40
    assert ipw % gb == 0

    mesh = plsc.VectorSubcoreMesh(core_axis_name="c", subcore_axis_name="s")

    @functools.partial(
        pl.kernel,
        out_type=jax.ShapeDtypeStruct((n_idx, F), jnp.float32),
        mesh=mesh,
        scratch_types=[
            pltpu.VMEM((ipw,), jnp.int32),
            pltpu.VMEM((gb, F), jnp.float32),
            pltpu.SemaphoreType.DMA,
        ],
    )
    def gat(table, idx, out, idx_v, rows_v, sem):
        c = lax.axis_index("c")
        t = lax.axis_index("s")
        wid = t * _NC + c
        base = wid * ipw
        pltpu.sync_copy(idx.at[pl.ds(base, ipw)], idx_v)

        def body(k, carry):
            pltpu.async_copy(table.at[idx_v.at[pl.ds(k * gb, gb)]],
                             rows_v, sem).wait()
            pltpu.sync_copy(rows_v, out.at[pl.ds(base + k * gb, gb)])
            return carry

        lax.fori_loop(0, ipw // gb, body, 0)

    return gat


def _unslice(raw, n_out, w):
    return raw.transpose(1, 0, 2).reshape(n_out, F)


def _d1_body(x_ref, deg_ref, z1_ref, z2_ref, pm_ref, wx, wd, w0, w1, wp,
             bias, pre_ref, s1_ref, s2_ref):
    i = pl.program_id(0)
    xb = x_ref[...]
    a = jnp.dot(xb, wx[...], preferred_element_type=jnp.float32)
    a += jnp.dot(xb * deg_ref[...], wd[...], preferred_element_type=jnp.float32)
    a += jnp.dot(z1_ref[...], w0[...], preferred_element_type=jnp.float32)
    a += jnp.dot(z2_ref[...], w1[...], preferred_element_type=jnp.float32)
    a += jnp.dot(pm_ref[...], wp[...], preferred_element_type=jnp.float32)
    a += bias[...]
    col = lax.broadcasted_iota(jnp.int32, a.shape, 1)
    a = jnp.where(col >= F // 2, jnp.maximum(a, 0.0), a)
    pre_ref[...] = a

    @pl.when(i == 0)
    def _init():
        s1_ref[...] = jnp.zeros_like(s1_ref)
        s2_ref[...] = jnp.zeros_like(s2_ref)

    s1_ref[...] += jnp.sum(a, axis=0, keepdims=True)
    s2_ref[...] += jnp.sum(a * a, axis=0, keepdims=True)


def _d2_body(pre_ref, s1_ref, s2_ref, gw, gb_, out_ref, *, n_rows):
    mean = s1_ref[...] / n_rows
    var = s2_ref[...] / n_rows - mean * mean
    inv = lax.rsqrt(var + 1e-5)
    out_ref[...] = (pre_ref[...] - mean) * inv * gw[...] + gb_[...]


def _dense_side(xx, deg, z1, z2, pm, wx, wd, w0, w1, wp, bias, gw, gb_, bs):
    n_rows = xx.shape[0]
    grid = n_rows // bs
    row = lambda i: (i, 0)
    const = lambda i: (0, 0)
    bspec = pl.BlockSpec((bs, F), row)
    wspec = pl.BlockSpec((F, F), const)
    sspec = pl.BlockSpec((1, F), const)

    pre, s1, s2 = pl.pallas_call(
        _d1_body,
        grid=(grid,),
        in_specs=[bspec, pl.BlockSpec((bs, 1), row), bspec, bspec, bspec,
                  wspec, wspec, wspec, wspec, wspec, sspec],
        out_specs=[bspec, sspec, sspec],
        out_shape=[jax.ShapeDtypeStruct((n_rows, F), jnp.float32),
                   jax.ShapeDtypeStruct((1, F), jnp.float32),
                   jax.ShapeDtypeStruct((1, F), jnp.float32)],
    )(xx, deg, z1, z2, pm, wx, wd, w0, w1, wp, bias)

    out = pl.pallas_call(
        functools.partial(_d2_body, n_rows=float(n_rows)),
        grid=(grid,),
        in_specs=[bspec, sspec, sspec, sspec, sspec],
        out_specs=bspec,
        out_shape=jax.ShapeDtypeStruct((n_rows, F), jnp.float32),
    )(pre, s1, s2, gw, gb_)
    return out


def _pad_idx(src, dst, n_pad_to, trash_row):
    npad = n_pad_to - src.shape[0]
    ar = jnp.arange(npad, dtype=src.dtype)
    srcp = jnp.concatenate([src, ar])
    dstp = jnp.concatenate([dst, trash_row + (ar % 8)])
    return srcp.reshape(-1, _MB), dstp.reshape(-1, _MB)


def kernel(x, y, deg_g, deg_lg, pm_pd, edge_index_g, edge_index_lg, params):
    p = params
    zeros_g = jnp.zeros((80, 64), jnp.float32)
    zeros_lg = jnp.zeros((1000, 8), jnp.float32)

    unit = _NT * _SB
    epg = ((E + unit - 1) // unit) * unit
    eplg = ((E_LG + unit - 1) // unit) * unit
    src_g, dst_g = _pad_idx(edge_index_g[0], edge_index_g[1], epg, 10240)
    src_lg, dst_lg = _pad_idx(edge_index_lg[0], edge_index_lg[1], eplg, E)

    seg_g1 = _make_segsum(epg, N, 64, 2, 2, 1, False, 0)
    seg_g2 = _make_segsum(epg, N, 64, 2, 1, N, False, 0)
    seg_gy = _make_segsum(epg, N, 64, 2, 2, 1, True, E - 1)
    seg_lg1 = _make_segsum(eplg, E, 8, 16, 16, 1, False, 0)
    seg_lg2 = _make_segsum(eplg, E, 8, 16, 1, E, False, 0)
    gather = _make_gather(E)

    z1g_r = seg_g1(x.reshape(N * 2, 64), src_g, dst_g, zeros_g)
    z2g_r = seg_g2(z1g_r.reshape(N * 2, 64), src_g, dst_g, zeros_g)
    pmy_r = seg_gy(y.reshape(E * 2, 64), src_g, dst_g, zeros_g)
    z1lg_r = seg_lg1(y.reshape(E * 16, 8), src_lg, dst_lg, zeros_lg)
    z2lg_r = seg_lg2(z1lg_r.reshape(E * 16, 8), src_lg, dst_lg, zeros_lg)
    pmx = gather(x, pm_pd)

    z1g = _unslice(z1g_r, N, 64)
    z2g = _unslice(z2g_r, N, 64)
    pmy = _unslice(pmy_r, N, 64)
    z1lg = _unslice(z1lg_r, E, 8)
    z2lg = _unslice(z2lg_r, E, 8)

    def wT(nm):
        return p[nm + '_w'].T

    bias_x = (p['theta_x_b'] + p['theta_deg_b'] + p['theta_0_b'] +
              p['theta_1_b'] + p['theta_y_b']).reshape(1, F)
    bias_y = (p['gamma_y_b'] + p['gamma_deg_b'] + p['gamma_0_b'] +
              p['gamma_1_b'] + p['gamma_x_b']).reshape(1, F)

    x_out = _dense_side(
        x, deg_g, z1g, z2g, pmy,
        wT('theta_x'), wT('theta_deg'), wT('theta_0'), wT('theta_1'),
        wT('theta_y'), bias_x,
        p['bn_x_w'].reshape(1, F), p['bn_x_b'].reshape(1, F), 1000)
    y_out = _dense_side(
        y, deg_lg, z1lg, z2lg, pmx,
        wT('gamma_y'), wT('gamma_deg'), wT('gamma_0'), wT('gamma_1'),
        wT('gamma_x'), bias_y,
        p['bn_y_w'].reshape(1, F), p['bn_y_b'].reshape(1, F), 2000)
    return (x_out, y_out)


# speedup vs baseline: 3.9888x; 1.0940x over previous
import functools

import jax
import jax.numpy as jnp
from jax import lax
from jax.experimental import pallas as pl
from jax.experimental.pallas import tpu as pltpu
from jax.experimental.pallas import tpu_sc as plsc

N = 10000
E = 160000
E_LG = 640000
F = 128

_NC = 2
_NT = 16
_MB = 128


@functools.lru_cache(maxsize=None)
def _make_segsum(n_edges, n_out, w, n_slices, src_mul, sl_mul, linear_src,
                 src_max):
    nmb = {64: 4, 8: 8}[w]
    sb = nmb * _MB
    ept = n_edges // _NT
    assert n_edges % (_NT * sb) == 0, n_edges
    nsb = ept // sb
    assert nsb % 2 == 0 and nsb >= 2
    zr = {64: 128, 8: 1000}[w]
    chunk = _NT * zr
    acc_rows = ((n_out + chunk - 1) // chunk) * chunk
    rpt = acc_rows // _NT
    assert rpt % zr == 0 and zr >= _MB
    spc = n_slices // _NC

    mesh = plsc.VectorSubcoreMesh(core_axis_name="c", subcore_axis_name="s")

    scratch = [
        [pltpu.VMEM((nmb, _MB), jnp.int32) for _ in range(2)],
        [pltpu.VMEM((nmb, _MB), jnp.int32) for _ in range(2)],
        [[pltpu.VMEM((_MB,), jnp.int32) for _ in range(nmb)]
         for _ in range(2)],
        [[pltpu.VMEM((_MB, w), jnp.float32) for _ in range(nmb)]
         for _ in range(2)],
        pltpu.VMEM((zr, w), jnp.float32),
        pltpu.VMEM_SHARED((acc_rows + 8, w), jnp.float32),
        pltpu.SemaphoreType.DMA,
        pltpu.SemaphoreType.DMA,
    ]

    @functools.partial(
        pl.kernel,
        out_type=jax.ShapeDtypeStruct((n_slices, n_out, w), jnp.float32),
        mesh=mesh,
        scratch_types=scratch,
        compiler_params=pltpu.CompilerParams(use_tc_tiling_on_sc=False),
    )
    def seg(table, src2, dst2, zeros, out, src_v, dst_v, gi, rows, z_v,
            acc, gsem, ssem):
        c = lax.axis_index("c")
        t = lax.axis_index("s")
        t_lo = t * (ept // _MB)
        row_lo = t * rpt
        n_valid = jnp.minimum(jnp.maximum(n_out - row_lo, 0), rpt)
        pltpu.sync_copy(zeros, z_v)

        def stage(kexpr, s, sl):
            rbase = t_lo + kexpr * nmb
            if not linear_src:
                pltpu.sync_copy(src2.at[pl.ds(rbase, nmb), :], src_v[s])
            pltpu.sync_copy(dst2.at[pl.ds(rbase, nmb), :], dst_v[s])
            for m in range(nmb):
                for u in range(_MB // 16):
                    if linear_src:
                        sv = ((rbase + m) * _MB + u * 16 +
                              lax.iota(jnp.int32, 16))
                        sv = jnp.minimum(sv, src_max)
                    else:
                        sv = src_v[s][m, pl.ds(u * 16, 16)]
                    gi[s][m][pl.ds(u * 16, 16)] = sv * src_mul + sl * sl_mul

        def fire_gathers(s):
            for m in range(nmb):
                pltpu.async_copy(table.at[gi[s][m]], rows[s][m], gsem)

        def wait_gathers(s):
            for m in range(nmb):
                pltpu.make_async_copy(table.at[gi[s][m]], rows[s][m],
                                      gsem).wait()

        def fire_scatters(s):
            for m in range(nmb):
                pltpu.async_copy(rows[s][m], acc.at[dst_v[s].at[m]], ssem,
                                 add=True)

        def drain_scatters(s):
            for m in range(nmb):
                pltpu.make_async_copy(zeros.at[pl.ds(0, _MB), :], rows[s][m],
                                      ssem).wait()

        for j in range(spc):
            sl = c * spc + j

            for q in range(rpt // zr):
                pltpu.sync_copy(z_v, acc.at[pl.ds(row_lo + q * zr, zr), :])
            plsc.subcore_barrier()

            stage(0, 0, sl)
            fire_gathers(0)
            for m in range(nmb):
                pltpu.async_copy(zeros.at[pl.ds(0, _MB), :], rows[1][m], ssem)

            def body(jj, carry):
                a = 2 * jj
                drain_scatters(1)
                stage(a + 1, 1, sl)
                fire_gathers(1)
                wait_gathers(0)
                fire_scatters(0)
                drain_scatters(0)
                stage(a + 2, 0, sl)
                fire_gathers(0)
                wait_gathers(1)
                fire_scatters(1)
                return carry

            lax.fori_loop(0, nsb // 2 - 1, body, 0)

            drain_scatters(1)
            stage(nsb - 1, 1, sl)
            fire_gathers(1)
            wait_gathers(0)
            fire_scatters(0)
            drain_scatters(0)
            wait_gathers(1)
            fire_scatters(1)
            drain_scatters(1)
            plsc.subcore_barrier()

            if acc_rows == n_out:
                pltpu.sync_copy(
                    acc.at[pl.ds(row_lo, rpt), :],
                    out.at[sl, pl.ds(row_lo, rpt), :])
            else:
                vrows = n_out - (_NT - 1) * rpt

                @pl.when(n_valid == rpt)
                def _():
                    pltpu.sync_copy(
                        acc.at[pl.ds(row_lo, rpt), :],
                        out.at[sl, pl.ds(row_lo, rpt), :])

                @pl.when(n_valid < rpt)
                def _():
                    pltpu.sync_copy(
                        acc.at[pl.ds(row_lo, vrows), :],
                        out.at[sl, pl.ds(row_lo, vrows), :])

            plsc.subcore_barrier()

    return seg


@functools.lru_cache(maxsize=None)
def _make_gather(n_idx):
    nw = _NC * _NT
    ipw = n_idx // nw
    gb = 40
    assert ipw % gb == 0

    mesh = plsc.VectorSubcoreMesh(core_axis_name="c", subcore_axis_name="s")

    @functools.partial(
        pl.kernel,
        out_type=jax.ShapeDtypeStruct((n_idx, F), jnp.float32),
        mesh=mesh,
        scratch_types=[
            pltpu.VMEM((ipw,), jnp.int32),
            pltpu.VMEM((gb, F), jnp.float32),
            pltpu.SemaphoreType.DMA,
        ],
    )
    def gat(table, idx, out, idx_v, rows_v, sem):
        c = lax.axis_index("c")
        t = lax.axis_index("s")
        wid = t * _NC + c
        base = wid * ipw
        pltpu.sync_copy(idx.at[pl.ds(base, ipw)], idx_v)

        def body(k, carry):
            pltpu.async_copy(table.at[idx_v.at[pl.ds(k * gb, gb)]],
                             rows_v, sem).wait()
            pltpu.sync_copy(rows_v, out.at[pl.ds(base + k * gb, gb)])
            return carry

        lax.fori_loop(0, ipw // gb, body, 0)

    return gat


def _unslice(raw, n_out, w):
    return raw.transpose(1, 0, 2).reshape(n_out, F)


def _d1_body(x_ref, deg_ref, z1_ref, z2_ref, pm_ref, wx, wd, w0, w1, wp,
             bias, pre_ref, s1_ref, s2_ref):
    i = pl.program_id(0)
    xb = x_ref[...]
    a = jnp.dot(xb, wx[...], preferred_element_type=jnp.float32)
    a += jnp.dot(xb * deg_ref[...], wd[...], preferred_element_type=jnp.float32)
    a += jnp.dot(z1_ref[...], w0[...], preferred_element_type=jnp.float32)
    a += jnp.dot(z2_ref[...], w1[...], preferred_element_type=jnp.float32)
    a += jnp.dot(pm_ref[...], wp[...], preferred_element_type=jnp.float32)
    a += bias[...]
    col = lax.broadcasted_iota(jnp.int32, a.shape, 1)
    a = jnp.where(col >= F // 2, jnp.maximum(a, 0.0), a)
    pre_ref[...] = a

    @pl.when(i == 0)
    def _init():
        s1_ref[...] = jnp.zeros_like(s1_ref)
        s2_ref[...] = jnp.zeros_like(s2_ref)

    s1_ref[...] += jnp.sum(a, axis=0, keepdims=True)
    s2_ref[...] += jnp.sum(a * a, axis=0, keepdims=True)


def _d2_body(pre_ref, s1_ref, s2_ref, gw, gb_, out_ref, *, n_rows):
    mean = s1_ref[...] / n_rows
    var = s2_ref[...] / n_rows - mean * mean
    inv = lax.rsqrt(var + 1e-5)
    out_ref[...] = (pre_ref[...] - mean) * inv * gw[...] + gb_[...]


def _dense_side(xx, deg, z1, z2, pm, wx, wd, w0, w1, wp, bias, gw, gb_, bs):
    n_rows = xx.shape[0]
    grid = n_rows // bs
    row = lambda i: (i, 0)
    const = lambda i: (0, 0)
    bspec = pl.BlockSpec((bs, F), row)
    wspec = pl.BlockSpec((F, F), const)
    sspec = pl.BlockSpec((1, F), const)

    pre, s1, s2 = pl.pallas_call(
        _d1_body,
        grid=(grid,),
        in_specs=[bspec, pl.BlockSpec((bs, 1), row), bspec, bspec, bspec,
                  wspec, wspec, wspec, wspec, wspec, sspec],
        out_specs=[bspec, sspec, sspec],
        out_shape=[jax.ShapeDtypeStruct((n_rows, F), jnp.float32),
                   jax.ShapeDtypeStruct((1, F), jnp.float32),
                   jax.ShapeDtypeStruct((1, F), jnp.float32)],
    )(xx, deg, z1, z2, pm, wx, wd, w0, w1, wp, bias)

    out = pl.pallas_call(
        functools.partial(_d2_body, n_rows=float(n_rows)),
        grid=(grid,),
        in_specs=[bspec, sspec, sspec, sspec, sspec],
        out_specs=bspec,
        out_shape=jax.ShapeDtypeStruct((n_rows, F), jnp.float32),
    )(pre, s1, s2, gw, gb_)
    return out


def _pad_idx(src, dst, n_pad_to, trash_row):
    npad = n_pad_to - src.shape[0]
    ar = jnp.arange(npad, dtype=src.dtype)
    srcp = jnp.concatenate([src, ar])
    dstp = jnp.concatenate([dst, trash_row + (ar % 8)])
    return srcp.reshape(-1, _MB), dstp.reshape(-1, _MB)


def kernel(x, y, deg_g, deg_lg, pm_pd, edge_index_g, edge_index_lg, params):
    p = params
    zeros_g = jnp.zeros((128, 64), jnp.float32)
    zeros_lg = jnp.zeros((1000, 8), jnp.float32)

    unit_g = _NT * 4 * _MB
    unit_lg = _NT * 8 * _MB
    epg = ((E + unit_g - 1) // unit_g) * unit_g
    eplg = ((E_LG + unit_lg - 1) // unit_lg) * unit_lg
    src_g, dst_g = _pad_idx(edge_index_g[0], edge_index_g[1], epg, 10240)
    src_lg, dst_lg = _pad_idx(edge_index_lg[0], edge_index_lg[1], eplg, E)

    seg_g1 = _make_segsum(epg, N, 64, 2, 2, 1, False, 0)
    seg_g2 = _make_segsum(epg, N, 64, 2, 1, N, False, 0)
    seg_gy = _make_segsum(epg, N, 64, 2, 2, 1, True, E - 1)
    seg_lg1 = _make_segsum(eplg, E, 8, 16, 16, 1, False, 0)
    seg_lg2 = _make_segsum(eplg, E, 8, 16, 1, E, False, 0)
    gather = _make_gather(E)

    z1g_r = seg_g1(x.reshape(N * 2, 64), src_g, dst_g, zeros_g)
    z2g_r = seg_g2(z1g_r.reshape(N * 2, 64), src_g, dst_g, zeros_g)
    pmy_r = seg_gy(y.reshape(E * 2, 64), src_g, dst_g, zeros_g)
    z1lg_r = seg_lg1(y.reshape(E * 16, 8), src_lg, dst_lg, zeros_lg)
    z2lg_r = seg_lg2(z1lg_r.reshape(E * 16, 8), src_lg, dst_lg, zeros_lg)
    pmx = gather(x, pm_pd)

    z1g = _unslice(z1g_r, N, 64)
    z2g = _unslice(z2g_r, N, 64)
    pmy = _unslice(pmy_r, N, 64)
    z1lg = _unslice(z1lg_r, E, 8)
    z2lg = _unslice(z2lg_r, E, 8)

    def wT(nm):
        return p[nm + '_w'].T

    bias_x = (p['theta_x_b'] + p['theta_deg_b'] + p['theta_0_b'] +
              p['theta_1_b'] + p['theta_y_b']).reshape(1, F)
    bias_y = (p['gamma_y_b'] + p['gamma_deg_b'] + p['gamma_0_b'] +
              p['gamma_1_b'] + p['gamma_x_b']).reshape(1, F)

    x_out = _dense_side(
        x, deg_g, z1g, z2g, pmy,
        wT('theta_x'), wT('theta_deg'), wT('theta_0'), wT('theta_1'),
        wT('theta_y'), bias_x,
        p['bn_x_w'].reshape(1, F), p['bn_x_b'].reshape(1, F), 1000)
    y_out = _dense_side(
        y, deg_lg, z1lg, z2lg, pmx,
        wT('gamma_y'), wT('gamma_deg'), wT('gamma_0'), wT('gamma_1'),
        wT('gamma_x'), bias_y,
        p['bn_y_w'].reshape(1, F), p['bn_y_b'].reshape(1, F), 2000)
    return (x_out, y_out)


# speedup vs baseline: 4.1333x; 1.0362x over previous
import functools

import jax
import jax.numpy as jnp
from jax import lax
from jax.experimental import pallas as pl
from jax.experimental.pallas import tpu as pltpu
from jax.experimental.pallas import tpu_sc as plsc

N = 10000
E = 160000
E_LG = 640000
F = 128

_NC = 2
_NT = 16
_MB = 128


@functools.lru_cache(maxsize=None)
def _make_segsum(n_edges, n_out, w, n_slices, src_mul, sl_mul, linear_src,
                 src_max):
    mb = {64: 256, 8: 512}[w]
    nmb = {64: 2, 8: 2}[w]
    sb = nmb * mb
    ept = n_edges // _NT
    assert n_edges % (_NT * sb) == 0, n_edges
    nsb = ept // sb
    assert nsb % 2 == 0 and nsb >= 2
    zr = {64: 128, 8: 1000}[w]
    zdim = max(zr, mb)
    chunk = _NT * zr
    acc_rows = ((n_out + chunk - 1) // chunk) * chunk
    rpt = acc_rows // _NT
    assert rpt % zr == 0
    spc = n_slices // _NC

    mesh = plsc.VectorSubcoreMesh(core_axis_name="c", subcore_axis_name="s")

    scratch = [
        [pltpu.VMEM((nmb, mb), jnp.int32) for _ in range(2)],
        [pltpu.VMEM((nmb, mb), jnp.int32) for _ in range(2)],
        [[pltpu.VMEM((mb,), jnp.int32) for _ in range(nmb)]
         for _ in range(2)],
        [[pltpu.VMEM((mb, w), jnp.float32) for _ in range(nmb)]
         for _ in range(2)],
        pltpu.VMEM((zdim, w), jnp.float32),
        pltpu.VMEM_SHARED((acc_rows + 8, w), jnp.float32),
        pltpu.SemaphoreType.DMA,
        pltpu.SemaphoreType.DMA,
    ]

    @functools.partial(
        pl.kernel,
        out_type=jax.ShapeDtypeStruct((n_slices, n_out, w), jnp.float32),
        mesh=mesh,
        scratch_types=scratch,
        compiler_params=pltpu.CompilerParams(use_tc_tiling_on_sc=False),
    )
    def seg(table, src2, dst2, zeros, out, src_v, dst_v, gi, rows, z_v,
            acc, gsem, ssem):
        c = lax.axis_index("c")
        t = lax.axis_index("s")
        t_lo = t * (ept // mb)
        row_lo = t * rpt
        n_valid = jnp.minimum(jnp.maximum(n_out - row_lo, 0), rpt)
        pltpu.sync_copy(zeros, z_v)

        def stage(kexpr, s, sl):
            rbase = t_lo + kexpr * nmb
            if not linear_src:
                pltpu.sync_copy(src2.at[pl.ds(rbase, nmb), :], src_v[s])
            pltpu.sync_copy(dst2.at[pl.ds(rbase, nmb), :], dst_v[s])
            for m in range(nmb):
                for u in range(mb // 16):
                    if linear_src:
                        sv = ((rbase + m) * mb + u * 16 +
                              lax.iota(jnp.int32, 16))
                        sv = jnp.minimum(sv, src_max)
                    else:
                        sv = src_v[s][m, pl.ds(u * 16, 16)]
                    gi[s][m][pl.ds(u * 16, 16)] = sv * src_mul + sl * sl_mul

        def fire_gathers(s):
            for m in range(nmb):
                pltpu.async_copy(table.at[gi[s][m]], rows[s][m], gsem)

        def wait_gathers(s):
            for m in range(nmb):
                pltpu.make_async_copy(table.at[gi[s][m]], rows[s][m],
                                      gsem).wait()

        def fire_scatters(s):
            for m in range(nmb):
                pltpu.async_copy(rows[s][m], acc.at[dst_v[s].at[m]], ssem,
                                 add=True)

        def drain_scatters(s):
            for m in range(nmb):
                pltpu.make_async_copy(zeros.at[pl.ds(0, mb), :], rows[s][m],
                                      ssem).wait()

        for j in range(spc):
            sl = c * spc + j

            for q in range(rpt // zr):
                pltpu.sync_copy(z_v.at[pl.ds(0, zr), :],
                                acc.at[pl.ds(row_lo + q * zr, zr), :])
            plsc.subcore_barrier()

            stage(0, 0, sl)
            fire_gathers(0)
            for m in range(nmb):
                pltpu.async_copy(zeros.at[pl.ds(0, mb), :], rows[1][m], ssem)

            def body(jj, carry):
                a = 2 * jj
                drain_scatters(1)
                stage(a + 1, 1, sl)
                fire_gathers(1)
                wait_gathers(0)
                fire_scatters(0)
                drain_scatters(0)
                stage(a + 2, 0, sl)
                fire_gathers(0)
                wait_gathers(1)
                fire_scatters(1)
                return carry

            lax.fori_loop(0, nsb // 2 - 1, body, 0)

            drain_scatters(1)
            stage(nsb - 1, 1, sl)
            fire_gathers(1)
            wait_gathers(0)
            fire_scatters(0)
            drain_scatters(0)
            wait_gathers(1)
            fire_scatters(1)
            drain_scatters(1)
            plsc.subcore_barrier()

            if acc_rows == n_out:
                pltpu.sync_copy(
                    acc.at[pl.ds(row_lo, rpt), :],
                    out.at[sl, pl.ds(row_lo, rpt), :])
            else:
                vpart = n_out % rpt
                tpart = n_out // rpt

                @pl.when(n_valid == rpt)
                def _():
                    pltpu.sync_copy(
                        acc.at[pl.ds(row_lo, rpt), :],
                        out.at[sl, pl.ds(row_lo, rpt), :])

                if vpart:
                    @pl.when(t == tpart)
                    def _():
                        pltpu.sync_copy(
                            acc.at[pl.ds(row_lo, vpart), :],
                            out.at[sl, pl.ds(row_lo, vpart), :])

            plsc.subcore_barrier()

    return seg


@functools.lru_cache(maxsize=None)
def _make_gather(n_idx):
    nw = _NC * _NT
    ipw = n_idx // nw
    gb = 40
    assert ipw % gb == 0

    mesh = plsc.VectorSubcoreMesh(core_axis_name="c", subcore_axis_name="s")

    @functools.partial(
        pl.kernel,
        out_type=jax.ShapeDtypeStruct((n_idx, F), jnp.float32),
        mesh=mesh,
        scratch_types=[
            pltpu.VMEM((ipw,), jnp.int32),
            pltpu.VMEM((gb, F), jnp.float32),
            pltpu.SemaphoreType.DMA,
        ],
    )
    def gat(table, idx, out, idx_v, rows_v, sem):
        c = lax.axis_index("c")
        t = lax.axis_index("s")
        wid = t * _NC + c
        base = wid * ipw
        pltpu.sync_copy(idx.at[pl.ds(base, ipw)], idx_v)

        def body(k, carry):
            pltpu.async_copy(table.at[idx_v.at[pl.ds(k * gb, gb)]],
                             rows_v, sem).wait()
            pltpu.sync_copy(rows_v, out.at[pl.ds(base + k * gb, gb)])
            return carry

        lax.fori_loop(0, ipw // gb, body, 0)

    return gat


def _unslice(raw, n_out, w):
    return raw.transpose(1, 0, 2).reshape(n_out, F)


def _d1_body(x_ref, deg_ref, z1_ref, z2_ref, pm_ref, wx, wd, w0, w1, wp,
             bias, pre_ref, s1_ref, s2_ref):
    i = pl.program_id(0)
    xb = x_ref[...]
    a = jnp.dot(xb, wx[...], preferred_element_type=jnp.float32)
    a += jnp.dot(xb * deg_ref[...], wd[...], preferred_element_type=jnp.float32)
    a += jnp.dot(z1_ref[...], w0[...], preferred_element_type=jnp.float32)
    a += jnp.dot(z2_ref[...], w1[...], preferred_element_type=jnp.float32)
    a += jnp.dot(pm_ref[...], wp[...], preferred_element_type=jnp.float32)
    a += bias[...]
    col = lax.broadcasted_iota(jnp.int32, a.shape, 1)
    a = jnp.where(col >= F // 2, jnp.maximum(a, 0.0), a)
    pre_ref[...] = a

    @pl.when(i == 0)
    def _init():
        s1_ref[...] = jnp.zeros_like(s1_ref)
        s2_ref[...] = jnp.zeros_like(s2_ref)

    s1_ref[...] += jnp.sum(a, axis=0, keepdims=True)
    s2_ref[...] += jnp.sum(a * a, axis=0, keepdims=True)


def _d2_body(pre_ref, s1_ref, s2_ref, gw, gb_, out_ref, *, n_rows):
    mean = s1_ref[...] / n_rows
    var = s2_ref[...] / n_rows - mean * mean
    inv = lax.rsqrt(var + 1e-5)
    out_ref[...] = (pre_ref[...] - mean) * inv * gw[...] + gb_[...]


def _dense_side(xx, deg, z1, z2, pm, wx, wd, w0, w1, wp, bias, gw, gb_, bs):
    n_rows = xx.shape[0]
    grid = n_rows // bs
    row = lambda i: (i, 0)
    const = lambda i: (0, 0)
    bspec = pl.BlockSpec((bs, F), row)
    wspec = pl.BlockSpec((F, F), const)
    sspec = pl.BlockSpec((1, F), const)

    pre, s1, s2 = pl.pallas_call(
        _d1_body,
        grid=(grid,),
        in_specs=[bspec, pl.BlockSpec((bs, 1), row), bspec, bspec, bspec,
                  wspec, wspec, wspec, wspec, wspec, sspec],
        out_specs=[bspec, sspec, sspec],
        out_shape=[jax.ShapeDtypeStruct((n_rows, F), jnp.float32),
                   jax.ShapeDtypeStruct((1, F), jnp.float32),
                   jax.ShapeDtypeStruct((1, F), jnp.float32)],
    )(xx, deg, z1, z2, pm, wx, wd, w0, w1, wp, bias)

    out = pl.pallas_call(
        functools.partial(_d2_body, n_rows=float(n_rows)),
        grid=(grid,),
        in_specs=[bspec, sspec, sspec, sspec, sspec],
        out_specs=bspec,
        out_shape=jax.ShapeDtypeStruct((n_rows, F), jnp.float32),
    )(pre, s1, s2, gw, gb_)
    return out


def _pad_idx(src, dst, n_pad_to, trash_row, mb):
    npad = n_pad_to - src.shape[0]
    ar = jnp.arange(npad, dtype=src.dtype)
    srcp = jnp.concatenate([src, ar])
    dstp = jnp.concatenate([dst, trash_row + (ar % 8)])
    return srcp.reshape(-1, mb), dstp.reshape(-1, mb)


def kernel(x, y, deg_g, deg_lg, pm_pd, edge_index_g, edge_index_lg, params):
    p = params
    zeros_g = jnp.zeros((256, 64), jnp.float32)
    zeros_lg = jnp.zeros((1000, 8), jnp.float32)

    unit_g = _NT * 2 * 256
    unit_lg = _NT * 2 * 512
    epg = ((E + unit_g - 1) // unit_g) * unit_g
    eplg = ((E_LG + unit_lg - 1) // unit_lg) * unit_lg
    src_g, dst_g = _pad_idx(edge_index_g[0], edge_index_g[1], epg, 10240, 256)
    src_lg, dst_lg = _pad_idx(edge_index_lg[0], edge_index_lg[1], eplg, E, 512)

    seg_g1 = _make_segsum(epg, N, 64, 2, 2, 1, False, 0)
    seg_g2 = _make_segsum(epg, N, 64, 2, 1, N, False, 0)
    seg_gy = _make_segsum(epg, N, 64, 2, 2, 1, True, E - 1)
    seg_lg1 = _make_segsum(eplg, E, 8, 16, 16, 1, False, 0)
    seg_lg2 = _make_segsum(eplg, E, 8, 16, 1, E, False, 0)
    gather = _make_gather(E)

    z1g_r = seg_g1(x.reshape(N * 2, 64), src_g, dst_g, zeros_g)
    z2g_r = seg_g2(z1g_r.reshape(N * 2, 64), src_g, dst_g, zeros_g)
    pmy_r = seg_gy(y.reshape(E * 2, 64), src_g, dst_g, zeros_g)
    z1lg_r = seg_lg1(y.reshape(E * 16, 8), src_lg, dst_lg, zeros_lg)
    z2lg_r = seg_lg2(z1lg_r.reshape(E * 16, 8), src_lg, dst_lg, zeros_lg)
    pmx = gather(x, pm_pd)

    z1g = _unslice(z1g_r, N, 64)
    z2g = _unslice(z2g_r, N, 64)
    pmy = _unslice(pmy_r, N, 64)
    z1lg = _unslice(z1lg_r, E, 8)
    z2lg = _unslice(z2lg_r, E, 8)

    def wT(nm):
        return p[nm + '_w'].T

    bias_x = (p['theta_x_b'] + p['theta_deg_b'] + p['theta_0_b'] +
              p['theta_1_b'] + p['theta_y_b']).reshape(1, F)
    bias_y = (p['gamma_y_b'] + p['gamma_deg_b'] + p['gamma_0_b'] +
              p['gamma_1_b'] + p['gamma_x_b']).reshape(1, F)

    x_out = _dense_side(
        x, deg_g, z1g, z2g, pmy,
        wT('theta_x'), wT('theta_deg'), wT('theta_0'), wT('theta_1'),
        wT('theta_y'), bias_x,
        p['bn_x_w'].reshape(1, F), p['bn_x_b'].reshape(1, F), 1000)
    y_out = _dense_side(
        y, deg_lg, z1lg, z2lg, pmx,
        wT('gamma_y'), wT('gamma_deg'), wT('gamma_0'), wT('gamma_1'),
        wT('gamma_x'), bias_y,
        p['bn_y_w'].reshape(1, F), p['bn_y_b'].reshape(1, F), 2000)
    return (x_out, y_out)


# speedup vs baseline: 4.5776x; 1.1075x over previous
import functools

import jax
import jax.numpy as jnp
from jax import lax
from jax.experimental import pallas as pl
from jax.experimental.pallas import tpu as pltpu
from jax.experimental.pallas import tpu_sc as plsc

N = 10000
E = 160000
E_LG = 640000
F = 128

_NC = 2
_NT = 16
_MB = 128


@functools.lru_cache(maxsize=None)
def _make_segsum(n_edges, n_out, w, n_slices, src_mul, sl_mul, linear_src,
                 src_max):
    mb = {64: 256, 8: 512}[w]
    nmb = {64: 2, 8: 2}[w]
    sb = nmb * mb
    ept = n_edges // _NT
    assert n_edges % (_NT * sb) == 0, n_edges
    nsb = ept // sb
    assert nsb % 2 == 0 and nsb >= 2
    zr = {64: 128, 8: 1000}[w]
    zdim = max(zr, mb)
    chunk = _NT * zr
    acc_rows = ((n_out + chunk - 1) // chunk) * chunk
    rpt = acc_rows // _NT
    assert rpt % zr == 0
    spc = n_slices // _NC

    mesh = plsc.VectorSubcoreMesh(core_axis_name="c", subcore_axis_name="s")

    scratch = [
        [pltpu.VMEM((nmb, mb), jnp.int32) for _ in range(2)],
        [pltpu.VMEM((nmb, mb), jnp.int32) for _ in range(2)],
        [[pltpu.VMEM((mb,), jnp.int32) for _ in range(nmb)]
         for _ in range(2)],
        [[pltpu.VMEM((mb, w), jnp.float32) for _ in range(nmb)]
         for _ in range(2)],
        pltpu.VMEM((zdim, w), jnp.float32),
        pltpu.VMEM_SHARED((acc_rows + 8, w), jnp.float32),
        pltpu.SemaphoreType.DMA,
        pltpu.SemaphoreType.DMA,
    ]

    @functools.partial(
        pl.kernel,
        out_type=jax.ShapeDtypeStruct((n_out, F), jnp.float32),
        mesh=mesh,
        scratch_types=scratch,
        compiler_params=pltpu.CompilerParams(use_tc_tiling_on_sc=False),
    )
    def seg(table, src2, dst2, zeros, out, src_v, dst_v, gi, rows, z_v,
            acc, gsem, ssem):
        c = lax.axis_index("c")
        t = lax.axis_index("s")
        t_lo = t * (ept // mb)
        row_lo = t * rpt
        n_valid = jnp.minimum(jnp.maximum(n_out - row_lo, 0), rpt)
        pltpu.sync_copy(zeros, z_v)

        def stage(kexpr, s, sl):
            rbase = t_lo + kexpr * nmb
            if not linear_src:
                pltpu.sync_copy(src2.at[pl.ds(rbase, nmb), :], src_v[s])
            pltpu.sync_copy(dst2.at[pl.ds(rbase, nmb), :], dst_v[s])
            for m in range(nmb):
                for u in range(mb // 16):
                    if linear_src:
                        sv = ((rbase + m) * mb + u * 16 +
                              lax.iota(jnp.int32, 16))
                        sv = jnp.where(sv > src_max, sv & 8191, sv)
                    else:
                        sv = src_v[s][m, pl.ds(u * 16, 16)]
                    gi[s][m][pl.ds(u * 16, 16)] = sv * src_mul + sl * sl_mul

        def fire_gathers(s):
            for m in range(nmb):
                pltpu.async_copy(table.at[gi[s][m]], rows[s][m], gsem)

        def wait_gathers(s):
            for m in range(nmb):
                pltpu.make_async_copy(table.at[gi[s][m]], rows[s][m],
                                      gsem).wait()

        def fire_scatters(s):
            for m in range(nmb):
                pltpu.async_copy(rows[s][m], acc.at[dst_v[s].at[m]], ssem,
                                 add=True)

        def drain_scatters(s):
            for m in range(nmb):
                pltpu.make_async_copy(zeros.at[pl.ds(0, mb), :], rows[s][m],
                                      ssem).wait()

        for j in range(spc):
            sl = c * spc + j

            for q in range(rpt // zr):
                pltpu.sync_copy(z_v.at[pl.ds(0, zr), :],
                                acc.at[pl.ds(row_lo + q * zr, zr), :])
            plsc.subcore_barrier()

            stage(0, 0, sl)
            fire_gathers(0)
            for m in range(nmb):
                pltpu.async_copy(zeros.at[pl.ds(0, mb), :], rows[1][m], ssem)

            def body(jj, carry):
                a = 2 * jj
                drain_scatters(1)
                stage(a + 1, 1, sl)
                fire_gathers(1)
                wait_gathers(0)
                fire_scatters(0)
                drain_scatters(0)
                stage(a + 2, 0, sl)
                fire_gathers(0)
                wait_gathers(1)
                fire_scatters(1)
                return carry

            lax.fori_loop(0, nsb // 2 - 1, body, 0)

            drain_scatters(1)
            stage(nsb - 1, 1, sl)
            fire_gathers(1)
            wait_gathers(0)
            fire_scatters(0)
            drain_scatters(0)
            wait_gathers(1)
            fire_scatters(1)
            drain_scatters(1)
            plsc.subcore_barrier()

            if acc_rows == n_out:
                pltpu.sync_copy(
                    acc.at[pl.ds(row_lo, rpt), :],
                    out.at[pl.ds(row_lo, rpt), pl.ds(sl * w, w)])
            else:
                vpart = n_out % rpt
                tpart = n_out // rpt

                @pl.when(n_valid == rpt)
                def _():
                    pltpu.sync_copy(
                        acc.at[pl.ds(row_lo, rpt), :],
                        out.at[pl.ds(row_lo, rpt), pl.ds(sl * w, w)])

                if vpart:
                    @pl.when(t == tpart)
                    def _():
                        pltpu.sync_copy(
                            acc.at[pl.ds(row_lo, vpart), :],
                            out.at[pl.ds(row_lo, vpart), pl.ds(sl * w, w)])

            plsc.subcore_barrier()

    return seg


@functools.lru_cache(maxsize=None)
def _make_gather(n_idx):
    nw = _NC * _NT
    ipw = n_idx // nw
    gb = 40
    assert ipw % gb == 0

    mesh = plsc.VectorSubcoreMesh(core_axis_name="c", subcore_axis_name="s")

    @functools.partial(
        pl.kernel,
        out_type=jax.ShapeDtypeStruct((n_idx, F), jnp.float32),
        mesh=mesh,
        scratch_types=[
            pltpu.VMEM((ipw,), jnp.int32),
            pltpu.VMEM((gb, F), jnp.float32),
            pltpu.SemaphoreType.DMA,
        ],
    )
    def gat(table, idx, out, idx_v, rows_v, sem):
        c = lax.axis_index("c")
        t = lax.axis_index("s")
        wid = t * _NC + c
        base = wid * ipw
        pltpu.sync_copy(idx.at[pl.ds(base, ipw)], idx_v)

        def body(k, carry):
            pltpu.async_copy(table.at[idx_v.at[pl.ds(k * gb, gb)]],
                             rows_v, sem).wait()
            pltpu.sync_copy(rows_v, out.at[pl.ds(base + k * gb, gb)])
            return carry

        lax.fori_loop(0, ipw // gb, body, 0)

    return gat


def _d1_body(x_ref, deg_ref, z1_ref, z2_ref, pm_ref, wx, wd, w0, w1, wp,
             bias, pre_ref, s1_ref, s2_ref):
    i = pl.program_id(0)
    xb = x_ref[...]
    a = jnp.dot(xb, wx[...], preferred_element_type=jnp.float32)
    a += jnp.dot(xb * deg_ref[...], wd[...], preferred_element_type=jnp.float32)
    a += jnp.dot(z1_ref[...], w0[...], preferred_element_type=jnp.float32)
    a += jnp.dot(z2_ref[...], w1[...], preferred_element_type=jnp.float32)
    a += jnp.dot(pm_ref[...], wp[...], preferred_element_type=jnp.float32)
    a += bias[...]
    col = lax.broadcasted_iota(jnp.int32, a.shape, 1)
    a = jnp.where(col >= F // 2, jnp.maximum(a, 0.0), a)
    pre_ref[...] = a

    @pl.when(i == 0)
    def _init():
        s1_ref[...] = jnp.zeros_like(s1_ref)
        s2_ref[...] = jnp.zeros_like(s2_ref)

    s1_ref[...] += jnp.sum(a, axis=0, keepdims=True)
    s2_ref[...] += jnp.sum(a * a, axis=0, keepdims=True)


def _d2_body(pre_ref, s1_ref, s2_ref, gw, gb_, out_ref, *, n_rows):
    mean = s1_ref[...] / n_rows
    var = s2_ref[...] / n_rows - mean * mean
    inv = lax.rsqrt(var + 1e-5)
    out_ref[...] = (pre_ref[...] - mean) * inv * gw[...] + gb_[...]


def _dense_side(xx, deg, z1, z2, pm, wx, wd, w0, w1, wp, bias, gw, gb_, bs):
    n_rows = xx.shape[0]
    grid = n_rows // bs
    row = lambda i: (i, 0)
    const = lambda i: (0, 0)
    bspec = pl.BlockSpec((bs, F), row)
    wspec = pl.BlockSpec((F, F), const)
    sspec = pl.BlockSpec((1, F), const)

    pre, s1, s2 = pl.pallas_call(
        _d1_body,
        grid=(grid,),
        in_specs=[bspec, pl.BlockSpec((bs, 1), row), bspec, bspec, bspec,
                  wspec, wspec, wspec, wspec, wspec, sspec],
        out_specs=[bspec, sspec, sspec],
        out_shape=[jax.ShapeDtypeStruct((n_rows, F), jnp.float32),
                   jax.ShapeDtypeStruct((1, F), jnp.float32),
                   jax.ShapeDtypeStruct((1, F), jnp.float32)],
    )(xx, deg, z1, z2, pm, wx, wd, w0, w1, wp, bias)

    out = pl.pallas_call(
        functools.partial(_d2_body, n_rows=float(n_rows)),
        grid=(grid,),
        in_specs=[bspec, sspec, sspec, sspec, sspec],
        out_specs=bspec,
        out_shape=jax.ShapeDtypeStruct((n_rows, F), jnp.float32),
    )(pre, s1, s2, gw, gb_)
    return out


def _pad_idx(src, dst, n_pad_to, trash_row, mb):
    npad = n_pad_to - src.shape[0]
    ar = jnp.arange(npad, dtype=src.dtype)
    srcp = jnp.concatenate([src, ar])
    dstp = jnp.concatenate([dst, trash_row + (ar % 8)])
    return srcp.reshape(-1, mb), dstp.reshape(-1, mb)


def kernel(x, y, deg_g, deg_lg, pm_pd, edge_index_g, edge_index_lg, params):
    p = params
    zeros_g = jnp.zeros((256, 64), jnp.float32)
    zeros_lg = jnp.zeros((1000, 8), jnp.float32)

    unit_g = _NT * 2 * 256
    unit_lg = _NT * 2 * 512
    epg = ((E + unit_g - 1) // unit_g) * unit_g
    eplg = ((E_LG + unit_lg - 1) // unit_lg) * unit_lg
    src_g, dst_g = _pad_idx(edge_index_g[0], edge_index_g[1], epg, 10240, 256)
    src_lg, dst_lg = _pad_idx(edge_index_lg[0], edge_index_lg[1], eplg, E, 512)

    seg_g = _make_segsum(epg, N, 64, 2, 2, 1, False, 0)
    seg_gy = _make_segsum(epg, N, 64, 2, 2, 1, True, E - 1)
    seg_lg = _make_segsum(eplg, E, 8, 16, 16, 1, False, 0)
    gather = _make_gather(E)

    z1g = seg_g(x.reshape(N * 2, 64), src_g, dst_g, zeros_g)
    z2g = seg_g(z1g.reshape(N * 2, 64), src_g, dst_g, zeros_g)
    pmy = seg_gy(y.reshape(E * 2, 64), src_g, dst_g, zeros_g)
    z1lg = seg_lg(y.reshape(E * 16, 8), src_lg, dst_lg, zeros_lg)
    z2lg = seg_lg(z1lg.reshape(E * 16, 8), src_lg, dst_lg, zeros_lg)
    pmx = gather(x, pm_pd)

    def wT(nm):
        return p[nm + '_w'].T

    bias_x = (p['theta_x_b'] + p['theta_deg_b'] + p['theta_0_b'] +
              p['theta_1_b'] + p['theta_y_b']).reshape(1, F)
    bias_y = (p['gamma_y_b'] + p['gamma_deg_b'] + p['gamma_0_b'] +
              p['gamma_1_b'] + p['gamma_x_b']).reshape(1, F)

    x_out = _dense_side(
        x, deg_g, z1g, z2g, pmy,
        wT('theta_x'), wT('theta_deg'), wT('theta_0'), wT('theta_1'),
        wT('theta_y'), bias_x,
        p['bn_x_w'].reshape(1, F), p['bn_x_b'].reshape(1, F), 1000)
    y_out = _dense_side(
        y, deg_lg, z1lg, z2lg, pmx,
        wT('gamma_y'), wT('gamma_deg'), wT('gamma_0'), wT('gamma_1'),
        wT('gamma_x'), bias_y,
        p['bn_y_w'].reshape(1, F), p['bn_y_b'].reshape(1, F), 2000)
    return (x_out, y_out)


# speedup vs baseline: 4.5986x; 1.0046x over previous
import functools

import jax
import jax.numpy as jnp
from jax import lax
from jax.experimental import pallas as pl
from jax.experimental.pallas import tpu as pltpu
from jax.experimental.pallas import tpu_sc as plsc

N = 10000
E = 160000
E_LG = 640000
F = 128

_NC = 2
_NT = 16
_MB = 128


@functools.lru_cache(maxsize=None)
def _make_segsum(n_edges, n_out, w, n_slices, src_mul, sl_mul, linear_src,
                 src_max):
    mb = {64: 256, 8: 1024}[w]
    nmb = {64: 2, 8: 1}[w]
    sb = nmb * mb
    ept = n_edges // _NT
    assert n_edges % (_NT * sb) == 0, n_edges
    nsb = ept // sb
    assert nsb % 2 == 0 and nsb >= 2
    zr = {64: 128, 8: 1000}[w]
    zdim = max(zr, mb)
    chunk = _NT * zr
    acc_rows = ((n_out + chunk - 1) // chunk) * chunk
    rpt = acc_rows // _NT
    assert rpt % zr == 0
    spc = n_slices // _NC

    mesh = plsc.VectorSubcoreMesh(core_axis_name="c", subcore_axis_name="s")

    scratch = [
        [pltpu.VMEM((nmb, mb), jnp.int32) for _ in range(2)],
        [pltpu.VMEM((nmb, mb), jnp.int32) for _ in range(2)],
        [[pltpu.VMEM((mb,), jnp.int32) for _ in range(nmb)]
         for _ in range(2)],
        [[pltpu.VMEM((mb, w), jnp.float32) for _ in range(nmb)]
         for _ in range(2)],
        pltpu.VMEM((zdim, w), jnp.float32),
        pltpu.VMEM_SHARED((acc_rows + 8, w), jnp.float32),
        pltpu.SemaphoreType.DMA,
        pltpu.SemaphoreType.DMA,
    ]

    @functools.partial(
        pl.kernel,
        out_type=jax.ShapeDtypeStruct((n_out, F), jnp.float32),
        mesh=mesh,
        scratch_types=scratch,
        compiler_params=pltpu.CompilerParams(use_tc_tiling_on_sc=False),
    )
    def seg(table, src2, dst2, zeros, out, src_v, dst_v, gi, rows, z_v,
            acc, gsem, ssem):
        c = lax.axis_index("c")
        t = lax.axis_index("s")
        t_lo = t * (ept // mb)
        row_lo = t * rpt
        n_valid = jnp.minimum(jnp.maximum(n_out - row_lo, 0), rpt)
        pltpu.sync_copy(zeros, z_v)

        def stage(kexpr, s, sl):
            rbase = t_lo + kexpr * nmb
            if not linear_src:
                pltpu.sync_copy(src2.at[pl.ds(rbase, nmb), :], src_v[s])
            pltpu.sync_copy(dst2.at[pl.ds(rbase, nmb), :], dst_v[s])
            for m in range(nmb):
                for u in range(mb // 16):
                    if linear_src:
                        sv = ((rbase + m) * mb + u * 16 +
                              lax.iota(jnp.int32, 16))
                        sv = jnp.where(sv > src_max, sv & 8191, sv)
                    else:
                        sv = src_v[s][m, pl.ds(u * 16, 16)]
                    gi[s][m][pl.ds(u * 16, 16)] = sv * src_mul + sl * sl_mul

        def fire_gathers(s):
            for m in range(nmb):
                pltpu.async_copy(table.at[gi[s][m]], rows[s][m], gsem)

        def wait_gathers(s):
            for m in range(nmb):
                pltpu.make_async_copy(table.at[gi[s][m]], rows[s][m],
                                      gsem).wait()

        def fire_scatters(s):
            for m in range(nmb):
                pltpu.async_copy(rows[s][m], acc.at[dst_v[s].at[m]], ssem,
                                 add=True)

        def drain_scatters(s):
            for m in range(nmb):
                pltpu.make_async_copy(zeros.at[pl.ds(0, mb), :], rows[s][m],
                                      ssem).wait()

        for j in range(spc):
            sl = c * spc + j

            for q in range(rpt // zr):
                pltpu.sync_copy(z_v.at[pl.ds(0, zr), :],
                                acc.at[pl.ds(row_lo + q * zr, zr), :])
            plsc.subcore_barrier()

            stage(0, 0, sl)
            fire_gathers(0)
            for m in range(nmb):
                pltpu.async_copy(zeros.at[pl.ds(0, mb), :], rows[1][m], ssem)

            def body(jj, carry):
                a = 2 * jj
                drain_scatters(1)
                stage(a + 1, 1, sl)
                fire_gathers(1)
                wait_gathers(0)
                fire_scatters(0)
                drain_scatters(0)
                stage(a + 2, 0, sl)
                fire_gathers(0)
                wait_gathers(1)
                fire_scatters(1)
                return carry

            lax.fori_loop(0, nsb // 2 - 1, body, 0)

            drain_scatters(1)
            stage(nsb - 1, 1, sl)
            fire_gathers(1)
            wait_gathers(0)
            fire_scatters(0)
            drain_scatters(0)
            wait_gathers(1)
            fire_scatters(1)
            drain_scatters(1)
            plsc.subcore_barrier()

            if acc_rows == n_out:
                pltpu.sync_copy(
                    acc.at[pl.ds(row_lo, rpt), :],
                    out.at[pl.ds(row_lo, rpt), pl.ds(sl * w, w)])
            else:
                vpart = n_out % rpt
                tpart = n_out // rpt

                @pl.when(n_valid == rpt)
                def _():
                    pltpu.sync_copy(
                        acc.at[pl.ds(row_lo, rpt), :],
                        out.at[pl.ds(row_lo, rpt), pl.ds(sl * w, w)])

                if vpart:
                    @pl.when(t == tpart)
                    def _():
                        pltpu.sync_copy(
                            acc.at[pl.ds(row_lo, vpart), :],
                            out.at[pl.ds(row_lo, vpart), pl.ds(sl * w, w)])

            plsc.subcore_barrier()

    return seg


@functools.lru_cache(maxsize=None)
def _make_gather(n_idx):
    nw = _NC * _NT
    ipw = n_idx // nw
    gb = 40
    assert ipw % gb == 0

    mesh = plsc.VectorSubcoreMesh(core_axis_name="c", subcore_axis_name="s")

    @functools.partial(
        pl.kernel,
        out_type=jax.ShapeDtypeStruct((n_idx, F), jnp.float32),
        mesh=mesh,
        scratch_types=[
            pltpu.VMEM((ipw,), jnp.int32),
            pltpu.VMEM((gb, F), jnp.float32),
            pltpu.SemaphoreType.DMA,
        ],
    )
    def gat(table, idx, out, idx_v, rows_v, sem):
        c = lax.axis_index("c")
        t = lax.axis_index("s")
        wid = t * _NC + c
        base = wid * ipw
        pltpu.sync_copy(idx.at[pl.ds(base, ipw)], idx_v)

        def body(k, carry):
            pltpu.async_copy(table.at[idx_v.at[pl.ds(k * gb, gb)]],
                             rows_v, sem).wait()
            pltpu.sync_copy(rows_v, out.at[pl.ds(base + k * gb, gb)])
            return carry

        lax.fori_loop(0, ipw // gb, body, 0)

    return gat


def _d1_body(x_ref, deg_ref, z1_ref, z2_ref, pm_ref, wx, wd, w0, w1, wp,
             bias, pre_ref, s1_ref, s2_ref):
    i = pl.program_id(0)
    xb = x_ref[...]
    a = jnp.dot(xb, wx[...], preferred_element_type=jnp.float32)
    a += jnp.dot(xb * deg_ref[...], wd[...], preferred_element_type=jnp.float32)
    a += jnp.dot(z1_ref[...], w0[...], preferred_element_type=jnp.float32)
    a += jnp.dot(z2_ref[...], w1[...], preferred_element_type=jnp.float32)
    a += jnp.dot(pm_ref[...], wp[...], preferred_element_type=jnp.float32)
    a += bias[...]
    col = lax.broadcasted_iota(jnp.int32, a.shape, 1)
    a = jnp.where(col >= F // 2, jnp.maximum(a, 0.0), a)
    pre_ref[...] = a

    @pl.when(i == 0)
    def _init():
        s1_ref[...] = jnp.zeros_like(s1_ref)
        s2_ref[...] = jnp.zeros_like(s2_ref)

    s1_ref[...] += jnp.sum(a, axis=0, keepdims=True)
    s2_ref[...] += jnp.sum(a * a, axis=0, keepdims=True)


def _d2_body(pre_ref, s1_ref, s2_ref, gw, gb_, out_ref, *, n_rows):
    mean = s1_ref[...] / n_rows
    var = s2_ref[...] / n_rows - mean * mean
    inv = lax.rsqrt(var + 1e-5)
    out_ref[...] = (pre_ref[...] - mean) * inv * gw[...] + gb_[...]


def _dense_side(xx, deg, z1, z2, pm, wx, wd, w0, w1, wp, bias, gw, gb_, bs):
    n_rows = xx.shape[0]
    grid = n_rows // bs
    row = lambda i: (i, 0)
    const = lambda i: (0, 0)
    bspec = pl.BlockSpec((bs, F), row)
    wspec = pl.BlockSpec((F, F), const)
    sspec = pl.BlockSpec((1, F), const)

    pre, s1, s2 = pl.pallas_call(
        _d1_body,
        grid=(grid,),
        in_specs=[bspec, pl.BlockSpec((bs, 1), row), bspec, bspec, bspec,
                  wspec, wspec, wspec, wspec, wspec, sspec],
        out_specs=[bspec, sspec, sspec],
        out_shape=[jax.ShapeDtypeStruct((n_rows, F), jnp.float32),
                   jax.ShapeDtypeStruct((1, F), jnp.float32),
                   jax.ShapeDtypeStruct((1, F), jnp.float32)],
    )(xx, deg, z1, z2, pm, wx, wd, w0, w1, wp, bias)

    out = pl.pallas_call(
        functools.partial(_d2_body, n_rows=float(n_rows)),
        grid=(grid,),
        in_specs=[bspec, sspec, sspec, sspec, sspec],
        out_specs=bspec,
        out_shape=jax.ShapeDtypeStruct((n_rows, F), jnp.float32),
    )(pre, s1, s2, gw, gb_)
    return out


def _pad_idx(src, dst, n_pad_to, trash_row, mb):
    npad = n_pad_to - src.shape[0]
    ar = jnp.arange(npad, dtype=src.dtype)
    srcp = jnp.concatenate([src, ar])
    dstp = jnp.concatenate([dst, trash_row + (ar % 8)])
    return srcp.reshape(-1, mb), dstp.reshape(-1, mb)


def kernel(x, y, deg_g, deg_lg, pm_pd, edge_index_g, edge_index_lg, params):
    p = params
    zeros_g = jnp.zeros((256, 64), jnp.float32)
    zeros_lg = jnp.zeros((1024, 8), jnp.float32)

    unit_g = _NT * 2 * 256
    unit_lg = _NT * 1 * 1024
    epg = ((E + unit_g - 1) // unit_g) * unit_g
    eplg = ((E_LG + unit_lg - 1) // unit_lg) * unit_lg
    src_g, dst_g = _pad_idx(edge_index_g[0], edge_index_g[1], epg, 10240, 256)
    src_lg, dst_lg = _pad_idx(edge_index_lg[0], edge_index_lg[1], eplg, E, 1024)

    seg_g = _make_segsum(epg, N, 64, 2, 2, 1, False, 0)
    seg_gy = _make_segsum(epg, N, 64, 2, 2, 1, True, E - 1)
    seg_lg = _make_segsum(eplg, E, 8, 16, 16, 1, False, 0)
    gather = _make_gather(E)

    z1g = seg_g(x.reshape(N * 2, 64), src_g, dst_g, zeros_g)
    z2g = seg_g(z1g.reshape(N * 2, 64), src_g, dst_g, zeros_g)
    pmy = seg_gy(y.reshape(E * 2, 64), src_g, dst_g, zeros_g)
    z1lg = seg_lg(y.reshape(E * 16, 8), src_lg, dst_lg, zeros_lg)
    z2lg = seg_lg(z1lg.reshape(E * 16, 8), src_lg, dst_lg, zeros_lg)
    pmx = gather(x, pm_pd)

    def wT(nm):
        return p[nm + '_w'].T

    bias_x = (p['theta_x_b'] + p['theta_deg_b'] + p['theta_0_b'] +
              p['theta_1_b'] + p['theta_y_b']).reshape(1, F)
    bias_y = (p['gamma_y_b'] + p['gamma_deg_b'] + p['gamma_0_b'] +
              p['gamma_1_b'] + p['gamma_x_b']).reshape(1, F)

    x_out = _dense_side(
        x, deg_g, z1g, z2g, pmy,
        wT('theta_x'), wT('theta_deg'), wT('theta_0'), wT('theta_1'),
        wT('theta_y'), bias_x,
        p['bn_x_w'].reshape(1, F), p['bn_x_b'].reshape(1, F), 1000)
    y_out = _dense_side(
        y, deg_lg, z1lg, z2lg, pmx,
        wT('gamma_y'), wT('gamma_deg'), wT('gamma_0'), wT('gamma_1'),
        wT('gamma_x'), bias_y,
        p['bn_y_w'].reshape(1, F), p['bn_y_b'].reshape(1, F), 2000)
    return (x_out, y_out)
